# Initial kernel scaffold; baseline (speedup 1.0000x reference)
import jax
import jax.numpy as jnp
from jax.experimental import pallas as pl


def kernel(signals, group_indices):
    raise NotImplementedError("write your pallas kernel here")


# speedup vs baseline: 1.5272x; 1.5272x over previous
import jax
import jax.numpy as jnp
from jax import lax
from jax.experimental import pallas as pl
from jax.experimental.pallas import tpu as pltpu
from jax.experimental.pallas import tpu_sc as plsc

_B, _C, _T = 16, 512, 4096
_G = 64
_CH = 64
_NS = 16
_NC = 2
_TH = _T // _NC
_NIT = _TH // _CH
_BS = _T // 4


def _sc_body(sig_ref, gidx_ref, out_ref,
             inbuf, outbuf, zbuf, ones, cntv, invv,
             idx0, idx1, idx2, idx3, acc, cnt_sh):
    cid = lax.axis_index("c")
    sid = lax.axis_index("s")
    b = sid
    base = sid * _G
    idxs = (idx0, idx1, idx2, idx3)

    for j, idx in enumerate(idxs):
        pltpu.sync_copy(gidx_ref.at[pl.ds(j * 128, 128)], idx)
    offv = jnp.full((16,), base, dtype=jnp.int32)
    for idx in idxs:
        for k in range(8):
            sl = pl.ds(k * 16, 16)
            idx[sl] = idx[sl] + offv

    zv = jnp.zeros((16,), jnp.float32)
    ov = jnp.ones((16,), jnp.float32)

    def _zrow(g, carry):
        for j in range(_CH // 16):
            zbuf[g, pl.ds(j * 16, 16)] = zv
        return carry

    lax.fori_loop(0, _G, _zrow, 0)

    def _orow(r, carry):
        ones[r, :] = ov
        return carry

    lax.fori_loop(0, 128, _orow, 0)

    pltpu.sync_copy(zbuf.at[:, pl.ds(0, 16)], cnt_sh.at[pl.ds(base, _G)])
    for idx in idxs:
        pltpu.sync_copy(ones, cnt_sh.at[idx], add=True)
    pltpu.sync_copy(cnt_sh.at[pl.ds(base, _G)], cntv)

    def _irow(g, carry):
        c = cntv[g, :]
        invv[g, :] = ov / jnp.maximum(c, ov)
        return carry

    lax.fori_loop(0, _G, _irow, 0)

    tbase = cid * _TH

    def _step(i, carry):
        t0 = tbase + i * _CH
        pltpu.sync_copy(sig_ref.at[pl.ds(b * _C, _C), pl.ds(t0, _CH)], inbuf)
        pltpu.sync_copy(zbuf, acc.at[pl.ds(base, _G)])
        for j, idx in enumerate(idxs):
            pltpu.sync_copy(inbuf.at[pl.ds(j * 128, 128)], acc.at[idx],
                            add=True)
        pltpu.sync_copy(acc.at[pl.ds(base, _G)], outbuf)

        def _srow(g, c2):
            iv = invv[g, :]
            for j in range(_CH // 16):
                sl = pl.ds(j * 16, 16)
                outbuf[g, sl] = outbuf[g, sl] * iv
            return c2

        lax.fori_loop(0, _G, _srow, 0)

        blk = t0 // _BS
        tin = t0 - blk * _BS
        orow = b * (4 * _G) + blk * _G
        pltpu.sync_copy(outbuf, out_ref.at[pl.ds(orow, _G), pl.ds(tin, _CH)])
        return carry

    lax.fori_loop(0, _NIT, _step, 0)


def kernel(signals, group_indices):
    Bv, Cv, Tv = signals.shape
    gi = group_indices.astype(jnp.int32)
    sig2d = signals.reshape(Bv * Cv, Tv)
    mesh = plsc.VectorSubcoreMesh(core_axis_name="c", subcore_axis_name="s")
    out2d = pl.kernel(
        _sc_body,
        out_type=jax.ShapeDtypeStruct((_B * 4 * _G, _BS), jnp.float32),
        mesh=mesh,
        compiler_params=pltpu.CompilerParams(use_tc_tiling_on_sc=False),
        scratch_types=[
            pltpu.VMEM((_C, _CH), jnp.float32),
            pltpu.VMEM((_G, _CH), jnp.float32),
            pltpu.VMEM((_G, _CH), jnp.float32),
            pltpu.VMEM((128, 16), jnp.float32),
            pltpu.VMEM((_G, 16), jnp.float32),
            pltpu.VMEM((_G, 16), jnp.float32),
            pltpu.VMEM((128,), jnp.int32),
            pltpu.VMEM((128,), jnp.int32),
            pltpu.VMEM((128,), jnp.int32),
            pltpu.VMEM((128,), jnp.int32),
            pltpu.VMEM_SHARED((_NS * _G, _CH), jnp.float32),
            pltpu.VMEM_SHARED((_NS * _G, 16), jnp.float32),
        ],
    )(sig2d, gi)
    return out2d.reshape(_B, 4 * _G, _BS)
